# baseline (speedup 1.0000x reference)
import functools

import jax
import jax.numpy as jnp
import numpy as np
from jax import lax
from jax.experimental import pallas as pl
from jax.experimental.pallas import tpu as pltpu
from jax.experimental.pallas import tpu_sc as plsc

_SELF_E = {1: -0.5, 6: -37.8, 7: -54.6, 8: -75.0, 9: -99.7,
           16: -398.1, 17: -460.1, 35: -2574.0, 99: -1000.0}

_N_ATOMS = 6_400_000
_M_MOL = 100_000
_MP = 102_400
_NC = 2
_NS = 16
_NW = _NC * _NS
_PER_W = _N_ATOMS // _NW
_B = 4_000
_NCHUNK = _PER_W // _B
_NBUF = 5
_NOUT = _NCHUNK // _NBUF
_F = 512
_NG = _B // 16
_M_SLICE = _MP // _NS
_ZW = 1_600


def _table_array():
    t = np.zeros((128,), dtype=np.float32)
    for z, v in _SELF_E.items():
        t[z] = v
    return jnp.asarray(t.view(np.int32))


def _sc_body(table_hbm, z_hbm, m_hbm, out_hbm,
             table_v, z_v, m_v, ce_v, cm_v, counts_v, offs_v,
             zero_v, accum_sh, in_sem, sc_sem):
    cid = lax.axis_index("c")
    sid = lax.axis_index("s")
    wid = sid * _NC + cid
    base = wid * _PER_W

    pltpu.sync_copy(table_hbm, table_v)

    for b in range(_NBUF):
        def _cm0(i, c, _b=b):
            cm_v[_b, pl.ds(i * 16, 16)] = jnp.zeros((16,), jnp.int32)
            return c
        lax.fori_loop(0, (_B + 16) // 16, _cm0, 0)

    def _ct0(i, c):
        counts_v[pl.ds(i * 16, 16)] = jnp.zeros((16,), jnp.int32)
        return c
    lax.fori_loop(0, 272 // 16, _ct0, 0)

    def _zl(i, c):
        zero_v[pl.ds(i * 16, 16)] = jnp.zeros((16,), jnp.float32)
        return c
    lax.fori_loop(0, _ZW // 16, _zl, 0)
    for t in range(_M_SLICE // _ZW):
        pltpu.sync_copy(zero_v,
                        accum_sh.at[pl.ds(sid * _M_SLICE + t * _ZW, _ZW)])
    plsc.subcore_barrier()

    def _in_desc(j, b):
        off = base + j * _B
        dz = pltpu.make_async_copy(z_hbm.at[pl.ds(off, _B)], z_v.at[b],
                                   in_sem.at[b])
        dm = pltpu.make_async_copy(m_hbm.at[pl.ds(off, _B)], m_v.at[b],
                                   in_sem.at[b])
        return dz, dm

    def _fire_in(j, b):
        dz, dm = _in_desc(j, b)
        dz.start()
        dm.start()

    def _full_desc(b):
        return pltpu.make_async_copy(ce_v.at[b, pl.ds(0, _B)],
                                     accum_sh.at[cm_v.at[b, pl.ds(0, _B)]],
                                     sc_sem.at[b])

    def _gather_compress(b):
        def _cz(i, c):
            ce_v[b, pl.ds(i * 16, 16)] = jnp.zeros((16,), jnp.float32)
            return c
        lax.fori_loop(0, _F // 16, _cz, 0)

        lane0 = lax.iota(jnp.int32, 16) == 0

        def _pa(i, c):
            z = z_v[b, pl.ds(i * 16, 16)]
            ei = plsc.load_gather(table_v, [z])
            mask = ei != 0
            pc = plsc.all_reduce_population_count(mask)
            plsc.store_compressed(counts_v.at[pl.ds(i, 16)], pc, mask=lane0)
            return c
        lax.fori_loop(0, _NG, _pa, 0, unroll=4)

        def _pb(t, car):
            c16 = counts_v[pl.ds(t * 16, 16)]
            cs = jnp.cumsum(c16)
            offs_v[pl.ds(t * 16, 16)] = cs - c16 + car
            return car + cs[15]
        total = lax.fori_loop(0, (_NG + 15) // 16, _pb, jnp.int32(0))

        def _pc(i, c):
            off = offs_v[pl.ds(i, 16)][0]
            z = z_v[b, pl.ds(i * 16, 16)]
            ei = plsc.load_gather(table_v, [z])
            mask = ei != 0
            e = plsc.bitcast(ei, jnp.float32)
            m = m_v[b, pl.ds(i * 16, 16)]
            plsc.store_compressed(ce_v.at[b, pl.ds(off, 16)], e, mask=mask)
            plsc.store_compressed(cm_v.at[b, pl.ds(off, 16)], m, mask=mask)
            return c
        lax.fori_loop(0, _NG, _pc, 0, unroll=4)
        return total

    def _cmp_desc(b):
        return pltpu.make_async_copy(ce_v.at[b, pl.ds(0, _F)],
                                     accum_sh.at[cm_v.at[b, pl.ds(0, _F)]],
                                     sc_sem.at[b])

    def _fire_scatter(b, off):
        @pl.when(off <= _F)
        def _compressed():
            _cmp_desc(b).start(add=True)

        @pl.when(off > _F)
        def _fallback():
            def _tz(k, c):
                ce_v[b, pl.ds(off + k * 16, 16)] = jnp.zeros((16,),
                                                             jnp.float32)
                return c
            lax.fori_loop(0, (_B - off + 15) // 16, _tz, 0)
            _full_desc(b).start(add=True)

    def _drain(b, off_b):
        @pl.when(off_b <= _F)
        def _wc():
            _cmp_desc(b).wait()

        @pl.when(off_b > _F)
        def _wf():
            _full_desc(b).wait()

    def _step(j, b, fire_j):
        dz, dm = _in_desc(j, b)
        dz.wait()
        dm.wait()
        off = _gather_compress(b)
        _fire_scatter(b, off)
        if fire_j is not None:
            _fire_in(fire_j, (b + 3) % _NBUF)
        return off

    for b in range(3):
        _fire_in(b, b)

    o1 = o2 = jnp.int32(0)
    for b in range(_NBUF):
        off = _step(b, b, fire_j=b + 3)
        if b >= 2:
            _drain((b - 2) % _NBUF, o2)
        o2, o1 = o1, off

    def _outer(o, carry):
        o1, o2 = carry
        for b in range(_NBUF):
            j = o * _NBUF + b
            off = _step(j, b, fire_j=j + 3)
            _drain((b - 2) % _NBUF, o2)
            o2, o1 = o1, off
        return o1, o2
    o1, o2 = lax.fori_loop(1, _NOUT - 1, _outer, (o1, o2))

    for b in range(_NBUF):
        j = (_NOUT - 1) * _NBUF + b
        off = _step(j, b,
                    fire_j=(j + 3) if (j + 3 < _NCHUNK) else None)
        _drain((b - 2) % _NBUF, o2)
        o2, o1 = o1, off

    _drain((_NBUF - 2) % _NBUF, o2)
    _drain((_NBUF - 1) % _NBUF, o1)

    plsc.subcore_barrier()
    pltpu.sync_copy(accum_sh.at[pl.ds(sid * _M_SLICE, _M_SLICE)],
                    out_hbm.at[cid, pl.ds(sid * _M_SLICE, _M_SLICE)])


def _segment_partials(table, atomic_numbers, molecule_indices):
    mesh = plsc.VectorSubcoreMesh(core_axis_name="c", subcore_axis_name="s")
    f = functools.partial(
        pl.kernel,
        mesh=mesh,
        compiler_params=pltpu.CompilerParams(needs_layout_passes=False,
                                             use_tc_tiling_on_sc=False),
        out_type=jax.ShapeDtypeStruct((_NC, _MP), jnp.float32),
        scratch_types=[
            pltpu.VMEM((128,), jnp.int32),
            pltpu.VMEM((_NBUF, _B), jnp.int32),
            pltpu.VMEM((_NBUF, _B), jnp.int32),
            pltpu.VMEM((_NBUF, _B + 16), jnp.float32),
            pltpu.VMEM((_NBUF, _B + 16), jnp.int32),
            pltpu.VMEM((272,), jnp.int32),
            pltpu.VMEM((272,), jnp.int32),
            pltpu.VMEM((_ZW,), jnp.float32),
            pltpu.VMEM_SHARED((_MP,), jnp.float32),
            pltpu.SemaphoreType.DMA((_NBUF,)),
            pltpu.SemaphoreType.DMA((_NBUF,)),
        ],
    )(_sc_body)
    return f(table, atomic_numbers, molecule_indices)


def _combine_body(e_ref, p_ref, o_ref):
    o_ref[...] = e_ref[...] + p_ref[0] + p_ref[1]


def _combine(energies_padded, partials):
    rows = _MP // 128
    out = pl.pallas_call(
        _combine_body,
        out_shape=jax.ShapeDtypeStruct((rows, 128), jnp.float32),
    )(energies_padded.reshape(rows, 128), partials.reshape(_NC, rows, 128))
    return out.reshape(_MP)


def kernel(energies, atomic_numbers, molecule_indices):
    table = _table_array()
    partials = _segment_partials(table, atomic_numbers, molecule_indices)
    e_pad = jnp.pad(energies, (0, _MP - _M_MOL))
    return _combine(e_pad, partials)[:_M_MOL]


# speedup vs baseline: 2.2976x; 2.2976x over previous
import functools

import jax
import jax.numpy as jnp
import numpy as np
from jax import lax
from jax.experimental import pallas as pl
from jax.experimental.pallas import tpu as pltpu
from jax.experimental.pallas import tpu_sc as plsc

_SELF_E = {1: -0.5, 6: -37.8, 7: -54.6, 8: -75.0, 9: -99.7,
           16: -398.1, 17: -460.1, 35: -2574.0, 99: -1000.0}

_N_ATOMS = 6_400_000
_M_MOL = 100_000
_MP = 102_400
_NC = 2
_NS = 16
_NW = _NC * _NS
_PER_W = _N_ATOMS // _NW
_B = 8_000
_NCHUNK = _PER_W // _B
_NBUF = 5
_NOUT = _NCHUNK // _NBUF
_M_SLICE = _MP // _NS
_ZW = 1_600


def _table_array():
    t = np.zeros((128,), dtype=np.float32)
    for z, v in _SELF_E.items():
        t[z] = v
    return jnp.asarray(t)


def _sc_body(table_hbm, z_hbm, m_hbm, out_hbm,
             table_v, z_v, m_v, e_v, zero_v, accum_sh, in_sem, sc_sem):
    cid = lax.axis_index("c")
    sid = lax.axis_index("s")
    wid = sid * _NC + cid
    base = wid * _PER_W

    pltpu.sync_copy(table_hbm, table_v)

    def _zl(i, c):
        zero_v[pl.ds(i * 16, 16)] = jnp.zeros((16,), jnp.float32)
        return c
    lax.fori_loop(0, _ZW // 16, _zl, 0)
    for t in range(_M_SLICE // _ZW):
        pltpu.sync_copy(zero_v,
                        accum_sh.at[pl.ds(sid * _M_SLICE + t * _ZW, _ZW)])
    plsc.subcore_barrier()

    def _in_desc(j, b):
        off = base + j * _B
        dz = pltpu.make_async_copy(z_hbm.at[pl.ds(off, _B)], z_v.at[b],
                                   in_sem.at[b])
        dm = pltpu.make_async_copy(m_hbm.at[pl.ds(off, _B)], m_v.at[b],
                                   in_sem.at[b])
        return dz, dm

    def _fire_in(j, b):
        dz, dm = _in_desc(j, b)
        dz.start()
        dm.start()

    def _sc_desc(b):
        return pltpu.make_async_copy(e_v.at[b], accum_sh.at[m_v.at[b]],
                                     sc_sem.at[b])

    def _gather(b):
        def _g(i, c):
            z = z_v[b, pl.ds(i * 16, 16)]
            e = plsc.load_gather(table_v, [z])
            e_v[b, pl.ds(i * 16, 16)] = e
            return c
        lax.fori_loop(0, _B // 16, _g, 0, unroll=8)

    def _step(j, b, fire_j):
        dz, dm = _in_desc(j, b)
        dz.wait()
        dm.wait()
        _gather(b)
        _sc_desc(b).start(add=True)
        if fire_j is not None:
            _fire_in(fire_j, (b + 3) % _NBUF)

    for b in range(3):
        _fire_in(b, b)

    for b in range(_NBUF):
        j = b
        dz, dm = _in_desc(j, b)
        dz.wait()
        dm.wait()
        _gather(b)
        _sc_desc(b).start(add=True)
        if j >= 2:
            _sc_desc((b - 2) % _NBUF).wait()
        _fire_in(j + 3, (b + 3) % _NBUF)

    def _outer(o, c):
        for b in range(_NBUF):
            j = o * _NBUF + b
            dz, dm = _in_desc(j, b)
            dz.wait()
            dm.wait()
            _gather(b)
            _sc_desc(b).start(add=True)
            _sc_desc((b - 2) % _NBUF).wait()
            _fire_in(j + 3, (b + 3) % _NBUF)
        return c
    lax.fori_loop(1, _NOUT - 1, _outer, 0)

    for b in range(_NBUF):
        j = (_NOUT - 1) * _NBUF + b
        dz, dm = _in_desc(j, b)
        dz.wait()
        dm.wait()
        _gather(b)
        _sc_desc(b).start(add=True)
        _sc_desc((b - 2) % _NBUF).wait()
        if j + 3 < _NCHUNK:
            _fire_in(j + 3, (b + 3) % _NBUF)

    _sc_desc((_NBUF - 2) % _NBUF).wait()
    _sc_desc((_NBUF - 1) % _NBUF).wait()

    plsc.subcore_barrier()
    pltpu.sync_copy(accum_sh.at[pl.ds(sid * _M_SLICE, _M_SLICE)],
                    out_hbm.at[cid, pl.ds(sid * _M_SLICE, _M_SLICE)])


def _segment_partials(table, atomic_numbers, molecule_indices):
    mesh = plsc.VectorSubcoreMesh(core_axis_name="c", subcore_axis_name="s")
    f = functools.partial(
        pl.kernel,
        mesh=mesh,
        compiler_params=pltpu.CompilerParams(needs_layout_passes=False,
                                             use_tc_tiling_on_sc=False),
        out_type=jax.ShapeDtypeStruct((_NC, _MP), jnp.float32),
        scratch_types=[
            pltpu.VMEM((128,), jnp.float32),
            pltpu.VMEM((_NBUF, _B), jnp.int32),
            pltpu.VMEM((_NBUF, _B), jnp.int32),
            pltpu.VMEM((_NBUF, _B), jnp.float32),
            pltpu.VMEM((_ZW,), jnp.float32),
            pltpu.VMEM_SHARED((_MP,), jnp.float32),
            pltpu.SemaphoreType.DMA((_NBUF,)),
            pltpu.SemaphoreType.DMA((_NBUF,)),
        ],
    )(_sc_body)
    return f(table, atomic_numbers, molecule_indices)


def _combine_body(e_ref, p_ref, o_ref):
    o_ref[...] = e_ref[...] + p_ref[0] + p_ref[1]


def _combine(energies_padded, partials):
    rows = _MP // 128
    out = pl.pallas_call(
        _combine_body,
        out_shape=jax.ShapeDtypeStruct((rows, 128), jnp.float32),
    )(energies_padded.reshape(rows, 128), partials.reshape(_NC, rows, 128))
    return out.reshape(_MP)


def kernel(energies, atomic_numbers, molecule_indices):
    table = _table_array()
    partials = _segment_partials(table, atomic_numbers, molecule_indices)
    e_pad = jnp.pad(energies, (0, _MP - _M_MOL))
    return _combine(e_pad, partials)[:_M_MOL]
